# baseline (speedup 1.0000x reference)
import functools

import jax
import jax.numpy as jnp
from jax import lax
from jax.experimental import pallas as pl
from jax.experimental.pallas import tpu as pltpu
from jax.experimental.pallas import tpu_sc as plsc

_N_ROWS = 4096
_N_COLS = 50
_B = _N_ROWS * _N_COLS
_D = 128
_BQ = _B // 4
_NC = 2
_NS = 16
_NW = _NC * _NS
_BPW = _BQ // _NW
_CH = 80
_NCHUNK = _BPW // _CH
_NBUF = 3

_mesh = plsc.VectorSubcoreMesh(core_axis_name="c", subcore_axis_name="s")


@functools.partial(
    pl.kernel,
    mesh=_mesh,
    out_type=jax.ShapeDtypeStruct((_B, _D), jnp.float32),
    scratch_types=[
        pltpu.VMEM((_BPW,), jnp.int32),
        pltpu.VMEM((_NBUF, _CH, 4, _D), jnp.float32),
        pltpu.SemaphoreType.DMA,
        pltpu.SemaphoreType.DMA,
        pltpu.SemaphoreType.DMA,
        pltpu.SemaphoreType.DMA,
        pltpu.SemaphoreType.DMA,
        pltpu.SemaphoreType.DMA,
    ],
)
def _embed_sc(qidx_hbm, qtable_hbm, out_hbm, idx_v, rows_v,
              g0, g1, g2, o0, o1, o2):
    rows_flat = rows_v.reshape(_NBUF, _CH * 4, _D)
    wid = lax.axis_index("s") * _NC + lax.axis_index("c")
    base = wid * _BPW
    pltpu.sync_copy(qidx_hbm.at[pl.ds(base, _BPW)], idx_v)

    gsems = (g0, g1, g2)
    osems = (o0, o1, o2)

    def start_gather(c):
        b = c % _NBUF
        return pltpu.async_copy(
            qtable_hbm.at[idx_v.at[pl.ds(c * _CH, _CH)]], rows_v.at[b],
            gsems[b])

    gather_pending = [None] * _NBUF
    out_pending = [None] * _NBUF
    for c in range(min(2, _NCHUNK)):
        gather_pending[c % _NBUF] = start_gather(c)
    for c in range(_NCHUNK):
        b = c % _NBUF
        gather_pending[b].wait()
        out_pending[b] = pltpu.async_copy(
            rows_flat.at[b],
            out_hbm.at[pl.ds((base + c * _CH) * 4, _CH * 4)], osems[b])
        n = c + 2
        if n < _NCHUNK:
            bn = n % _NBUF
            if out_pending[bn] is not None:
                out_pending[bn].wait()
                out_pending[bn] = None
            gather_pending[bn] = start_gather(n)
    for b in range(_NBUF):
        if out_pending[b] is not None:
            out_pending[b].wait()


def kernel(cue, table):
    idx = cue.reshape(_BQ, 4).astype(jnp.int32)
    qidx = ((idx[:, 0] * 4 + idx[:, 1]) * 4 + idx[:, 2]) * 4 + idx[:, 3]
    digits = (jnp.arange(256, dtype=jnp.int32)[:, None]
              // jnp.array([64, 16, 4, 1], dtype=jnp.int32)) % 4
    qtable = jnp.take(table.astype(jnp.float32), digits, axis=0)
    out = _embed_sc(qidx, qtable)
    return out.reshape(_N_ROWS, _N_COLS, _D)


# speedup vs baseline: 1.4730x; 1.4730x over previous
import functools

import jax
import jax.numpy as jnp
from jax import lax
from jax.experimental import pallas as pl
from jax.experimental.pallas import tpu as pltpu
from jax.experimental.pallas import tpu_sc as plsc

_N_ROWS = 4096
_N_COLS = 50
_B = _N_ROWS * _N_COLS
_D = 128
_BQ = _B // 4
_NC = 2
_NS = 16
_NW = _NC * _NS
_RPW = _N_ROWS // _NW
_CR = 8
_CH = _CR * _N_COLS // 4
_BPW = _BQ // _NW
_NCHUNK = _RPW // _CR
_NBUF = 2
_CHP = 104
_IPW = _NCHUNK * _CHP

_mesh = plsc.VectorSubcoreMesh(core_axis_name="c", subcore_axis_name="s")


@functools.partial(
    pl.kernel,
    mesh=_mesh,
    out_type=jax.ShapeDtypeStruct((_N_ROWS, _N_COLS, _D), jnp.float32),
    scratch_types=[
        pltpu.VMEM((_IPW,), jnp.int32),
        pltpu.VMEM((_NBUF, _CH, 4, _D), jnp.float32),
        pltpu.SemaphoreType.DMA,
        pltpu.SemaphoreType.DMA,
        pltpu.SemaphoreType.DMA,
        pltpu.SemaphoreType.DMA,
    ],
)
def _embed_sc(qidx_hbm, qtable_hbm, out_hbm, idx_v, rows_q, g0, g1, o0, o1):
    rows_v = rows_q.reshape(_NBUF, _CR, _N_COLS, _D)
    wid = lax.axis_index("s") * _NC + lax.axis_index("c")
    rbase = wid * _RPW
    pltpu.sync_copy(qidx_hbm.at[pl.ds(wid * _IPW, _IPW)], idx_v)

    gsems = (g0, g1)
    osems = (o0, o1)

    def start_gather(c):
        b = c % _NBUF
        return pltpu.async_copy(
            qtable_hbm.at[idx_v.at[pl.ds(c * _CHP, _CH)]], rows_q.at[b],
            gsems[b])

    gather_pending = [None] * _NBUF
    out_pending = [None] * _NBUF
    gather_pending[0] = start_gather(0)
    for c in range(_NCHUNK):
        b = c % _NBUF
        gather_pending[b].wait()
        out_pending[b] = pltpu.async_copy(
            rows_v.at[b], out_hbm.at[pl.ds(rbase + c * _CR, _CR)], osems[b])
        n = c + 1
        if n < _NCHUNK:
            bn = n % _NBUF
            if out_pending[bn] is not None:
                out_pending[bn].wait()
                out_pending[bn] = None
            gather_pending[bn] = start_gather(n)
    for b in range(_NBUF):
        if out_pending[b] is not None:
            out_pending[b].wait()


def kernel(cue, table):
    idx = cue.reshape(_BQ, 4).astype(jnp.int32)
    qidx = ((idx[:, 0] * 4 + idx[:, 1]) * 4 + idx[:, 2]) * 4 + idx[:, 3]
    qidx = jnp.pad(qidx.reshape(_NW, _NCHUNK, _CH),
                   ((0, 0), (0, 0), (0, _CHP - _CH))).reshape(-1)
    digits = (jnp.arange(256, dtype=jnp.int32)[:, None]
              // jnp.array([64, 16, 4, 1], dtype=jnp.int32)) % 4
    qtable = jnp.take(table.astype(jnp.float32), digits, axis=0)
    return _embed_sc(qidx, qtable)
